# Initial kernel scaffold; baseline (speedup 1.0000x reference)
import jax
import jax.numpy as jnp
from jax.experimental import pallas as pl


def kernel(g, g_base, x, road_ID, etype_base, road_embed, poi_distribution, time_distribution, road_feature, poi_W, poi_b, time_W, time_b, road_W, road_b, comb1, V1, loop1, b1, comb2, V2, loop2, b2):
    raise NotImplementedError("write your pallas kernel here")


# speedup vs baseline: 9.7253x; 9.7253x over previous
import functools

import jax
import jax.numpy as jnp
from jax import lax
from jax.experimental import pallas as pl
from jax.experimental.pallas import tpu as pltpu
from jax.experimental.pallas import tpu_sc as plsc

_N = 10000
_E = 320000
_R = 8
_NB = 8
_HID = 128

_NC = 2
_NS = 16
_NW = _NC * _NS
_CH = 128
_K = -(-_E // (_NW * _CH))
_SPAN = _K * _CH
_EP = _NW * _SPAN
_ROWS_PER_TILE = 632
_NPAD = _NS * _ROWS_PER_TILE

_BLK = 400
_NBLK = _N // _BLK


def _feat_body(poi_ref, time_ref, road_ref, poiW_ref, poib_ref,
               timeW_ref, timeb_ref, roadW_ref, roadb_ref, out_ref):
    a = jnp.dot(poi_ref[...], poiW_ref[...],
                preferred_element_type=jnp.float32) + poib_ref[...]
    b = jnp.dot(time_ref[...], timeW_ref[...],
                preferred_element_type=jnp.float32) + timeb_ref[...]
    c = jnp.dot(road_ref[...], roadW_ref[...],
                preferred_element_type=jnp.float32) + roadb_ref[...]
    out_ref[...] = jnp.concatenate([a, b, c], axis=1)


def _feat_call(poi, time, road, poiW, poib, timeW, timeb, roadW, roadb):
    full = lambda shape: pl.BlockSpec(shape, lambda i: (0, 0))
    return pl.pallas_call(
        _feat_body,
        grid=(_NBLK,),
        in_specs=[
            pl.BlockSpec((_BLK, poi.shape[1]), lambda i: (i, 0)),
            pl.BlockSpec((_BLK, time.shape[1]), lambda i: (i, 0)),
            pl.BlockSpec((_BLK, road.shape[1]), lambda i: (i, 0)),
            full(poiW.shape), full(poib.shape),
            full(timeW.shape), full(timeb.shape),
            full(roadW.shape), full(roadb.shape),
        ],
        out_specs=pl.BlockSpec((_BLK, _HID), lambda i: (i, 0)),
        out_shape=jax.ShapeDtypeStruct((_N, _HID), jnp.float32),
    )(poi, time, road, poiW, poib, timeW, timeb, roadW, roadb)


def _w_body(comb_ref, V_ref, out_ref):
    r = pl.program_id(0)

    def body(b, acc):
        return acc + comb_ref[r, b] * V_ref[b]

    out_ref[0] = lax.fori_loop(0, _NB + 1, body,
                               jnp.zeros((_HID, _HID), jnp.float32))


def _w_call(comb_ext, V_ext):
    return pl.pallas_call(
        _w_body,
        grid=(_R + 1,),
        in_specs=[
            pl.BlockSpec(memory_space=pltpu.SMEM),
            pl.BlockSpec((_R + 1, _HID, _HID), lambda r: (0, 0, 0)),
        ],
        out_specs=pl.BlockSpec((1, _HID, _HID), lambda r: (r, 0, 0)),
        out_shape=jax.ShapeDtypeStruct((_R + 1, _HID, _HID), jnp.float32),
    )(comb_ext, V_ext)


def _h_body(W_ref, feat_ref, out_ref):
    out_ref[0] = jnp.dot(feat_ref[...], W_ref[0],
                         preferred_element_type=jnp.float32)


def _h_call(Wfull, feat):
    return pl.pallas_call(
        _h_body,
        grid=(_R + 1, _NBLK),
        in_specs=[
            pl.BlockSpec((1, _HID, _HID), lambda r, i: (r, 0, 0)),
            pl.BlockSpec((_BLK, _HID), lambda r, i: (i, 0)),
        ],
        out_specs=pl.BlockSpec((1, _BLK, _HID), lambda r, i: (r, i, 0)),
        out_shape=jax.ShapeDtypeStruct((_R + 1, _N, _HID), jnp.float32),
    )(Wfull, feat)


def _combine_body(part_ref, self_ref, bias_ref, out_ref, *, relu):
    h = part_ref[0] + part_ref[1] + self_ref[...] + bias_ref[...]
    out_ref[...] = jnp.maximum(h, 0.0) if relu else h


def _combine_call(part, self_msg, bias, relu):
    return pl.pallas_call(
        functools.partial(_combine_body, relu=relu),
        grid=(_NBLK,),
        in_specs=[
            pl.BlockSpec((_NC, _BLK, _HID), lambda i: (0, i, 0)),
            pl.BlockSpec((_BLK, _HID), lambda i: (i, 0)),
            pl.BlockSpec((1, _HID), lambda i: (0, 0)),
        ],
        out_specs=pl.BlockSpec((_BLK, _HID), lambda i: (i, 0)),
        out_shape=jax.ShapeDtypeStruct((_N, _HID), jnp.float32),
    )(part, self_msg, bias)


@functools.partial(
    pl.kernel,
    out_type=jax.ShapeDtypeStruct((_NC, _NPAD, _HID), jnp.float32),
    mesh=plsc.VectorSubcoreMesh(core_axis_name="c", subcore_axis_name="s"),
    scratch_types=[
        pltpu.VMEM((_CH,), jnp.int32),
        pltpu.VMEM((_CH,), jnp.int32),
        pltpu.VMEM((_CH, _HID), jnp.float32),
        pltpu.VMEM_SHARED((_NPAD, _HID), jnp.float32),
        pltpu.SemaphoreType.DMA,
    ],
)
def _sc_agg(h_hbm, gidx_hbm, dst_hbm, zeros_hbm, out_hbm,
            idx_v, dst_v, rows_v, agg_s, sem):
    c = lax.axis_index("c")
    s = lax.axis_index("s")
    wid = s * _NC + c
    row0 = s * _ROWS_PER_TILE
    pltpu.sync_copy(zeros_hbm.at[pl.ds(row0, _ROWS_PER_TILE)],
                    agg_s.at[pl.ds(row0, _ROWS_PER_TILE)])
    plsc.subcore_barrier()
    base = wid * _SPAN

    def body(i, carry):
        off = base + i * _CH
        pltpu.sync_copy(gidx_hbm.at[pl.ds(off, _CH)], idx_v)
        pltpu.sync_copy(dst_hbm.at[pl.ds(off, _CH)], dst_v)
        pltpu.async_copy(h_hbm.at[idx_v], rows_v, sem).wait()
        pltpu.sync_copy(rows_v, agg_s.at[dst_v], add=True)
        return carry

    lax.fori_loop(0, _K, body, 0)
    plsc.subcore_barrier()
    pltpu.sync_copy(agg_s.at[pl.ds(row0, _ROWS_PER_TILE)],
                    out_hbm.at[c, pl.ds(row0, _ROWS_PER_TILE)])


def kernel(g, g_base, x, road_ID, etype_base, road_embed, poi_distribution,
           time_distribution, road_feature, poi_W, poi_b, time_W, time_b,
           road_W, road_b, comb1, V1, loop1, b1, comb2, V2, loop2, b2):
    src = g_base[0].astype(jnp.int32)
    dst = g_base[1].astype(jnp.int32)
    et = etype_base.astype(jnp.int32)
    gidx = et * _N + src
    pad = _EP - _E
    gidx_p = jnp.concatenate([gidx, jnp.zeros((pad,), jnp.int32)])
    dst_p = jnp.concatenate([dst, jnp.full((pad,), _N, jnp.int32)])
    zeros = jnp.zeros((_NPAD, _HID), jnp.float32)

    feat = _feat_call(poi_distribution, time_distribution, road_feature,
                      poi_W, poi_b.reshape(1, -1), time_W,
                      time_b.reshape(1, -1), road_W, road_b.reshape(1, -1))

    def layer(f, comb, V, loop_w, bias, relu):
        comb_ext = (jnp.zeros((_R + 1, _NB + 1), jnp.float32)
                    .at[:_R, :_NB].set(comb).at[_R, _NB].set(1.0))
        V_ext = jnp.concatenate([V, loop_w[None]], axis=0)
        Wfull = _w_call(comb_ext, V_ext)
        Hfull = _h_call(Wfull, f)
        table = Hfull.reshape((_R + 1) * _N, _HID)
        part = _sc_agg(table, gidx_p, dst_p, zeros)
        return _combine_call(part, Hfull[_R], bias.reshape(1, -1), relu)

    h = layer(feat, comb1, V1, loop1, b1, True)
    return layer(h, comb2, V2, loop2, b2, False)
